# baseline (speedup 1.0000x reference)
import functools

import jax
from jax import numpy as jnp
from jax import lax
from jax.experimental import pallas as pl
from jax.experimental.pallas import tpu as pltpu
from jax.experimental.pallas import tpu_sc as plsc

N = 100000
D = 128
C = 2
E = 3200000

NC = 2
NS = 16
LANE = 128

W = 8

N_PAD = 102400
R_TOT = E // LANE
SG = 16
MAIN_ROWS = R_TOT // NS // SG * SG


def _project_body(x_ref, w_ref, b_ref, o_ref):
    o_ref[...] = (
        jnp.dot(x_ref[...], w_ref[...], preferred_element_type=jnp.float32)
        + b_ref[...])


def _project(feat, w8, b8):
    br = 4000
    return pl.pallas_call(
        _project_body,
        grid=(N // br,),
        in_specs=[
            pl.BlockSpec((br, D), lambda i: (i, 0)),
            pl.BlockSpec((D, W), lambda i: (0, 0)),
            pl.BlockSpec((1, W), lambda i: (0, 0)),
        ],
        out_specs=pl.BlockSpec((br, W), lambda i: (i, 0)),
        out_shape=jax.ShapeDtypeStruct((N, W), jnp.float32),
    )(feat, w8, b8)


def _vtake(x, idx):
    dnums = lax.GatherDimensionNumbers(
        offset_dims=(), collapsed_slice_dims=(0,), start_index_map=(0,))
    return lax.gather(x, idx[:, None], dnums, (1,),
                      mode=lax.GatherScatterMode.PROMISE_IN_BOUNDS)


def _sc_aggregate(wm_rel, wm_rev, edges_rel, edges_rev, zeros):
    mesh = plsc.VectorSubcoreMesh(core_axis_name="c", subcore_axis_name="s")
    rpt = N_PAD // NS

    @functools.partial(
        pl.kernel,
        out_type=jax.ShapeDtypeStruct((2, N_PAD, W), jnp.float32),
        mesh=mesh,
        compiler_params=pltpu.CompilerParams(use_tc_tiling_on_sc=False),
        scratch_types=[
            pltpu.VMEM_SHARED((N, W), jnp.float32),
            pltpu.VMEM_SHARED((N_PAD, W), jnp.float32),
            pltpu.VMEM((SG, LANE), jnp.int32),
            pltpu.VMEM((SG, LANE), jnp.int32),
            pltpu.VMEM((SG, LANE, W), jnp.float32),
            pltpu.SemaphoreType.DMA,
            pltpu.SemaphoreType.DMA,
        ])
    def sc_kernel(wm_rel_hbm, wm_rev_hbm, edges_rel_hbm, edges_rev_hbm,
                  zeros_hbm, dump_hbm,
                  table, acc, srcv, dstv, msg, gsem, ssem):
        cid = lax.axis_index("c")
        sid = lax.axis_index("s")

        def run(wm_hbm, edges_hbm, dump_plane):
            r0 = sid * rpt
            t0 = sid * (N // NS)
            pltpu.sync_copy(wm_hbm.at[pl.ds(t0, N // NS)],
                            table.at[pl.ds(t0, N // NS)])
            pltpu.sync_copy(zeros_hbm.at[pl.ds(r0, rpt)], acc.at[pl.ds(r0, rpt)])
            plsc.subcore_barrier()

            start = R_TOT * sid // NS
            end = R_TOT * (sid + 1) // NS

            @pl.loop(0, MAIN_ROWS, step=SG)
            def _(g):
                row = start + g
                pltpu.sync_copy(edges_hbm.at[0, pl.ds(row, SG)], srcv)
                pltpu.sync_copy(edges_hbm.at[1, pl.ds(row, SG)], dstv)
                gcps = [
                    pltpu.async_copy(table.at[srcv.at[j]], msg.at[j], gsem)
                    for j in range(SG)
                ]
                scps = []
                for j in range(SG):
                    gcps[j].wait()
                    scps.append(pltpu.async_copy(
                        msg.at[j], acc.at[dstv.at[j]], ssem, add=True))
                for s in scps:
                    s.wait()

            @pl.loop(start + MAIN_ROWS, end)
            def _(row):
                pltpu.sync_copy(edges_hbm.at[0, pl.ds(row, 1)],
                                srcv.at[pl.ds(0, 1)])
                pltpu.sync_copy(edges_hbm.at[1, pl.ds(row, 1)],
                                dstv.at[pl.ds(0, 1)])
                pltpu.async_copy(table.at[srcv.at[0]], msg.at[0], gsem).wait()
                pltpu.sync_copy(msg.at[0], acc.at[dstv.at[0]], add=True)

            plsc.subcore_barrier()

            pltpu.sync_copy(acc.at[pl.ds(r0, rpt)],
                            dump_plane.at[pl.ds(r0, rpt)])

        @pl.when(cid == 0)
        def _():
            run(wm_rel_hbm, edges_rel_hbm, dump_hbm.at[0])

        @pl.when(cid == 1)
        def _():
            run(wm_rev_hbm, edges_rev_hbm, dump_hbm.at[1])

    return sc_kernel(wm_rel, wm_rev, edges_rel, edges_rev, zeros)


def _sc_mean(acc_flat):
    mesh = plsc.VectorSubcoreMesh(core_axis_name="c", subcore_axis_name="s")
    rpt = N_PAD // NS
    ovr = rpt * C // 128

    @functools.partial(
        pl.kernel,
        out_type=jax.ShapeDtypeStruct((2, N_PAD * C // 128, 128), jnp.float32),
        mesh=mesh,
        compiler_params=pltpu.CompilerParams(use_tc_tiling_on_sc=False),
        scratch_types=[
            pltpu.VMEM((rpt * W,), jnp.float32),
            pltpu.VMEM((ovr, 128), jnp.float32),
        ])
    def mean_kernel(acc_hbm, out_hbm, av, ov):
        cid = lax.axis_index("c")
        sid = lax.axis_index("s")
        i16 = lax.iota(jnp.int32, 16)
        dpat = (i16 >> 3) * 8 + 2
        cpat = ((i16 >> 1) & 1) * 8 + (i16 & 1)

        def run(plane, out_plane):
            r0 = sid * rpt
            pltpu.sync_copy(plane.at[pl.ds(r0 * W, rpt * W)], av)

            @pl.loop(0, rpt * W, step=8 * 64)
            def _(i):
                for t in range(8):
                    q = []
                    for k in range(4):
                        v = av[pl.ds(i + 64 * t + 16 * k, 16)]
                        dvec = _vtake(v, dpat)
                        qk = jnp.where(dvec > 0.0,
                                       v / jnp.maximum(dvec, 1.0), 0.0)
                        q.append(_vtake(qk, cpat))
                    out = jnp.where(i16 < 4, q[0],
                                    jnp.where(i16 < 8, q[1],
                                              jnp.where(i16 < 12,
                                                        q[2], q[3])))
                    ov[i >> 9, pl.ds(16 * t, 16)] = out

            pltpu.sync_copy(ov, out_plane.at[pl.ds(sid * ovr, ovr)])

        @pl.when(cid == 0)
        def _():
            run(acc_hbm.at[0], out_hbm.at[1])

        @pl.when(cid == 1)
        def _():
            run(acc_hbm.at[1], out_hbm.at[0])

    return mean_kernel(acc_flat)


def _finish(packed):
    mesh = plsc.VectorSubcoreMesh(core_axis_name="c", subcore_axis_name="s")
    rpt = N_PAD // NS
    last = N - (NS - 1) * rpt

    @functools.partial(
        pl.kernel,
        out_type=jax.ShapeDtypeStruct((2, N, C), jnp.float32),
        mesh=mesh,
        compiler_params=pltpu.CompilerParams(use_tc_tiling_on_sc=False),
        scratch_types=[])
    def fin_kernel(src_hbm, out_hbm):
        cid = lax.axis_index("c")
        sid = lax.axis_index("s")
        p = src_hbm.at[cid]
        o = out_hbm.at[cid]
        r0 = sid * rpt

        @pl.when(sid < NS - 1)
        def _():
            pltpu.sync_copy(p.at[pl.ds(r0, rpt)], o.at[pl.ds(r0, rpt)])

        @pl.when(sid == NS - 1)
        def _():
            pltpu.sync_copy(p.at[pl.ds(r0, last)], o.at[pl.ds(r0, last)])

    return fin_kernel(packed)


def kernel(feat_user, feat_item, edge_index_rel, edge_index_rev,
           W_rel, b_rel, W_rev, b_rev):
    wz = jnp.zeros((D, W - C), jnp.float32)
    tail = jnp.concatenate([jnp.ones((1,), jnp.float32),
                            jnp.zeros((W - C - 1,), jnp.float32)])
    w8_rel = jnp.concatenate([W_rel, wz], axis=1)
    w8_rev = jnp.concatenate([W_rev, wz], axis=1)
    b8_rel = jnp.concatenate([b_rel, tail]).reshape(1, W)
    b8_rev = jnp.concatenate([b_rev, tail]).reshape(1, W)

    wm_rel = _project(feat_user, w8_rel, b8_rel)
    wm_rev = _project(feat_item, w8_rev, b8_rev)

    edges_rel = edge_index_rel.reshape(2, R_TOT, LANE)
    edges_rev = edge_index_rev.reshape(2, R_TOT, LANE)

    zeros = jnp.zeros((N_PAD, W), jnp.float32)
    dump = _sc_aggregate(wm_rel, wm_rev, edges_rel, edges_rev, zeros)
    out = _sc_mean(dump.reshape(2, N_PAD * W))
    return _finish(out.reshape(2, N_PAD, C))


# speedup vs baseline: 1.7465x; 1.7465x over previous
import functools

import jax
from jax import numpy as jnp
from jax import lax
from jax.experimental import pallas as pl
from jax.experimental.pallas import tpu as pltpu
from jax.experimental.pallas import tpu_sc as plsc

N = 100000
D = 128
C = 2
E = 3200000

NC = 2
NS = 16
LANE = 128

W = 8

N_PAD = 100096
R_TOT = E // LANE
SG = 16
MAIN_ROWS = R_TOT // NS // SG * SG


def _project_body(x_ref, w_ref, b_ref, o_ref):
    o_ref[...] = (
        jnp.dot(x_ref[...], w_ref[...], preferred_element_type=jnp.float32)
        + b_ref[...])


def _project(feat, w8, b8):
    br = 4000
    return pl.pallas_call(
        _project_body,
        grid=(N // br,),
        in_specs=[
            pl.BlockSpec((br, D), lambda i: (i, 0)),
            pl.BlockSpec((D, W), lambda i: (0, 0)),
            pl.BlockSpec((1, W), lambda i: (0, 0)),
        ],
        out_specs=pl.BlockSpec((br, W), lambda i: (i, 0)),
        out_shape=jax.ShapeDtypeStruct((N, W), jnp.float32),
    )(feat, w8, b8)


def _vtake(x, idx):
    dnums = lax.GatherDimensionNumbers(
        offset_dims=(), collapsed_slice_dims=(0,), start_index_map=(0,))
    return lax.gather(x, idx[:, None], dnums, (1,),
                      mode=lax.GatherScatterMode.PROMISE_IN_BOUNDS)


def _sc_aggregate(wm_rel, wm_rev, edges_rel, edges_rev, zeros):
    mesh = plsc.VectorSubcoreMesh(core_axis_name="c", subcore_axis_name="s")
    rpt = N_PAD // NS

    @functools.partial(
        pl.kernel,
        out_type=jax.ShapeDtypeStruct((2, N_PAD, W), jnp.float32),
        mesh=mesh,
        compiler_params=pltpu.CompilerParams(use_tc_tiling_on_sc=False),
        scratch_types=[
            pltpu.VMEM_SHARED((N, W), jnp.float32),
            pltpu.VMEM_SHARED((N_PAD, W), jnp.float32),
            pltpu.VMEM((2, SG, LANE), jnp.int32),
            pltpu.VMEM((2, SG, LANE), jnp.int32),
            pltpu.VMEM((SG, LANE, W), jnp.float32),
            pltpu.SemaphoreType.DMA,
            pltpu.SemaphoreType.DMA,
            pltpu.SemaphoreType.DMA,
        ])
    def sc_kernel(wm_rel_hbm, wm_rev_hbm, edges_rel_hbm, edges_rev_hbm,
                  zeros_hbm, dump_hbm,
                  table, acc, srcv, dstv, msg, gsem, ssem, esem):
        cid = lax.axis_index("c")
        sid = lax.axis_index("s")

        def run(wm_hbm, edges_hbm, dump_plane):
            r0 = sid * rpt
            t0 = sid * (N // NS)
            pltpu.sync_copy(wm_hbm.at[pl.ds(t0, N // NS)],
                            table.at[pl.ds(t0, N // NS)])
            pltpu.sync_copy(zeros_hbm.at[pl.ds(r0, rpt)], acc.at[pl.ds(r0, rpt)])
            plsc.subcore_barrier()

            start = R_TOT * sid // NS
            end = R_TOT * (sid + 1) // NS

            pltpu.async_copy(edges_hbm.at[0, pl.ds(start, SG)],
                             srcv.at[0], esem)
            pltpu.async_copy(edges_hbm.at[1, pl.ds(start, SG)],
                             dstv.at[0], esem)
            pltpu.async_copy(edges_hbm.at[0, pl.ds(start + SG, SG)],
                             srcv.at[1], esem)
            pltpu.async_copy(edges_hbm.at[1, pl.ds(start + SG, SG)],
                             dstv.at[1], esem)

            @pl.loop(0, MAIN_ROWS, step=SG)
            def _(g):
                b = (g // SG) & 1
                row = start + g
                pltpu.make_async_copy(edges_hbm.at[0, pl.ds(row, SG)],
                                      srcv.at[b], esem).wait()
                pltpu.make_async_copy(edges_hbm.at[1, pl.ds(row, SG)],
                                      dstv.at[b], esem).wait()
                gcps = [
                    pltpu.async_copy(table.at[srcv.at[b, j]], msg.at[j], gsem)
                    for j in range(SG)
                ]
                scps = []
                for j in range(SG):
                    gcps[j].wait()
                    scps.append(pltpu.async_copy(
                        msg.at[j], acc.at[dstv.at[b, j]], ssem, add=True))
                for s in scps:
                    s.wait()
                nrow = row + 2 * SG

                @pl.when(nrow + SG <= start + MAIN_ROWS)
                def _():
                    pltpu.async_copy(edges_hbm.at[0, pl.ds(nrow, SG)],
                                     srcv.at[b], esem)
                    pltpu.async_copy(edges_hbm.at[1, pl.ds(nrow, SG)],
                                     dstv.at[b], esem)

            @pl.loop(start + MAIN_ROWS, end)
            def _(row):
                pltpu.sync_copy(edges_hbm.at[0, pl.ds(row, 1)],
                                srcv.at[0, pl.ds(0, 1)])
                pltpu.sync_copy(edges_hbm.at[1, pl.ds(row, 1)],
                                dstv.at[0, pl.ds(0, 1)])
                pltpu.async_copy(table.at[srcv.at[0, 0]], msg.at[0],
                                 gsem).wait()
                pltpu.sync_copy(msg.at[0], acc.at[dstv.at[0, 0]], add=True)

            plsc.subcore_barrier()

            pltpu.sync_copy(acc.at[pl.ds(r0, rpt)],
                            dump_plane.at[pl.ds(r0, rpt)])

        @pl.when(cid == 0)
        def _():
            run(wm_rel_hbm, edges_rel_hbm, dump_hbm.at[0])

        @pl.when(cid == 1)
        def _():
            run(wm_rev_hbm, edges_rev_hbm, dump_hbm.at[1])

    return sc_kernel(wm_rel, wm_rev, edges_rel, edges_rev, zeros)


def _sc_mean(acc_flat):
    mesh = plsc.VectorSubcoreMesh(core_axis_name="c", subcore_axis_name="s")
    rpt = N_PAD // NS
    ovr = rpt * C // 16

    @functools.partial(
        pl.kernel,
        out_type=jax.ShapeDtypeStruct((2, N * C // 16, 16), jnp.float32),
        mesh=mesh,
        compiler_params=pltpu.CompilerParams(use_tc_tiling_on_sc=False),
        scratch_types=[
            pltpu.VMEM((rpt * W,), jnp.float32),
            pltpu.VMEM((ovr, 16), jnp.float32),
        ])
    def mean_kernel(acc_hbm, out_hbm, av, ov):
        cid = lax.axis_index("c")
        sid = lax.axis_index("s")
        i16 = lax.iota(jnp.int32, 16)
        dpat = (i16 >> 3) * 8 + 2
        cpat = ((i16 >> 1) & 1) * 8 + (i16 & 1)

        def run(plane, out_plane):
            r0 = sid * rpt
            pltpu.sync_copy(plane.at[pl.ds(r0 * W, rpt * W)], av)

            @pl.loop(0, rpt * W, step=4 * 16)
            def _(i):
                q = []
                for k in range(4):
                    v = av[pl.ds(i + 16 * k, 16)]
                    dvec = _vtake(v, dpat)
                    qk = jnp.where(dvec > 0.0,
                                   v / jnp.maximum(dvec, 1.0), 0.0)
                    q.append(_vtake(qk, cpat))
                out = jnp.where(i16 < 4, q[0],
                                jnp.where(i16 < 8, q[1],
                                          jnp.where(i16 < 12, q[2], q[3])))
                ov[i >> 6] = out

            last = (N - (NS - 1) * rpt) * C // 16
            o0 = r0 * C // 16

            @pl.when(sid < NS - 1)
            def _():
                pltpu.sync_copy(ov, out_plane.at[pl.ds(o0, ovr)])

            @pl.when(sid == NS - 1)
            def _():
                pltpu.sync_copy(ov.at[pl.ds(0, last)],
                                out_plane.at[pl.ds(o0, last)])

        @pl.when(cid == 0)
        def _():
            run(acc_hbm.at[0], out_hbm.at[1])

        @pl.when(cid == 1)
        def _():
            run(acc_hbm.at[1], out_hbm.at[0])

    return mean_kernel(acc_flat)


def kernel(feat_user, feat_item, edge_index_rel, edge_index_rev,
           W_rel, b_rel, W_rev, b_rev):
    wz = jnp.zeros((D, W - C), jnp.float32)
    tail = jnp.concatenate([jnp.ones((1,), jnp.float32),
                            jnp.zeros((W - C - 1,), jnp.float32)])
    w8_rel = jnp.concatenate([W_rel, wz], axis=1)
    w8_rev = jnp.concatenate([W_rev, wz], axis=1)
    b8_rel = jnp.concatenate([b_rel, tail]).reshape(1, W)
    b8_rev = jnp.concatenate([b_rev, tail]).reshape(1, W)

    wm_rel = _project(feat_user, w8_rel, b8_rel)
    wm_rev = _project(feat_item, w8_rev, b8_rev)

    edges_rel = edge_index_rel.reshape(2, R_TOT, LANE)
    edges_rev = edge_index_rev.reshape(2, R_TOT, LANE)

    zeros = jnp.zeros((N_PAD, W), jnp.float32)
    dump = _sc_aggregate(wm_rel, wm_rev, edges_rel, edges_rev, zeros)
    out = _sc_mean(dump.reshape(2, N_PAD * W))
    return out.reshape(2, N, C)


# speedup vs baseline: 1.8698x; 1.0706x over previous
import functools

import jax
from jax import numpy as jnp
from jax import lax
from jax.experimental import pallas as pl
from jax.experimental.pallas import tpu as pltpu
from jax.experimental.pallas import tpu_sc as plsc

N = 100000
D = 128
C = 2
E = 3200000

NC = 2
NS = 16
LANE = 128

W = 8

N_PAD = 100096
R_TOT = E // LANE
SG = 16
MAIN_ROWS = R_TOT // NS // SG * SG


def _project_body(x_ref, w_ref, b_ref, o_ref):
    o_ref[...] = (
        jnp.dot(x_ref[...], w_ref[...], preferred_element_type=jnp.float32)
        + b_ref[...])


def _project(feat, w8, b8):
    br = 4000
    return pl.pallas_call(
        _project_body,
        grid=(N // br,),
        in_specs=[
            pl.BlockSpec((br, D), lambda i: (i, 0)),
            pl.BlockSpec((D, W), lambda i: (0, 0)),
            pl.BlockSpec((1, W), lambda i: (0, 0)),
        ],
        out_specs=pl.BlockSpec((br, W), lambda i: (i, 0)),
        out_shape=jax.ShapeDtypeStruct((N, W), jnp.float32),
    )(feat, w8, b8)


def _vtake(x, idx):
    dnums = lax.GatherDimensionNumbers(
        offset_dims=(), collapsed_slice_dims=(0,), start_index_map=(0,))
    return lax.gather(x, idx[:, None], dnums, (1,),
                      mode=lax.GatherScatterMode.PROMISE_IN_BOUNDS)


def _sc_aggregate(wm_rel, wm_rev, edges_rel, edges_rev, zeros):
    mesh = plsc.VectorSubcoreMesh(core_axis_name="c", subcore_axis_name="s")
    rpt = N_PAD // NS

    @functools.partial(
        pl.kernel,
        out_type=jax.ShapeDtypeStruct((2, N_PAD, W), jnp.float32),
        mesh=mesh,
        compiler_params=pltpu.CompilerParams(use_tc_tiling_on_sc=False),
        scratch_types=[
            pltpu.VMEM_SHARED((N, W), jnp.float32),
            pltpu.VMEM_SHARED((N_PAD, W), jnp.float32),
            pltpu.VMEM((2, SG, LANE), jnp.int32),
            pltpu.VMEM((2, SG, LANE), jnp.int32),
            pltpu.VMEM((SG, LANE, W), jnp.float32),
            pltpu.SemaphoreType.DMA,
            pltpu.SemaphoreType.DMA,
            pltpu.SemaphoreType.DMA,
        ])
    def sc_kernel(wm_rel_hbm, wm_rev_hbm, edges_rel_hbm, edges_rev_hbm,
                  zeros_hbm, dump_hbm,
                  table, acc, srcv, dstv, msg, gsem, ssem, esem):
        cid = lax.axis_index("c")
        sid = lax.axis_index("s")

        def run(wm_hbm, edges_hbm, dump_plane):
            r0 = sid * rpt
            t0 = sid * (N // NS)
            pltpu.sync_copy(wm_hbm.at[pl.ds(t0, N // NS)],
                            table.at[pl.ds(t0, N // NS)])
            pltpu.sync_copy(zeros_hbm.at[pl.ds(r0, rpt)], acc.at[pl.ds(r0, rpt)])
            plsc.subcore_barrier()

            start = R_TOT * sid // NS
            end = R_TOT * (sid + 1) // NS

            pltpu.async_copy(edges_hbm.at[0, pl.ds(start, SG)],
                             srcv.at[0], esem)
            pltpu.async_copy(edges_hbm.at[1, pl.ds(start, SG)],
                             dstv.at[0], esem)
            pltpu.async_copy(edges_hbm.at[0, pl.ds(start + SG, SG)],
                             srcv.at[1], esem)
            pltpu.async_copy(edges_hbm.at[1, pl.ds(start + SG, SG)],
                             dstv.at[1], esem)

            pltpu.make_async_copy(edges_hbm.at[0, pl.ds(start, SG)],
                                  srcv.at[0], esem).wait()
            pltpu.make_async_copy(edges_hbm.at[1, pl.ds(start, SG)],
                                  dstv.at[0], esem).wait()
            for j in range(SG):
                pltpu.async_copy(table.at[srcv.at[0, j]], msg.at[j], gsem)

            @pl.loop(0, MAIN_ROWS, step=SG)
            def _(g):
                b = (g // SG) & 1
                nb = b ^ 1
                row = start + g

                @pl.when(g + SG < MAIN_ROWS)
                def _():
                    pltpu.make_async_copy(
                        edges_hbm.at[0, pl.ds(row + SG, SG)],
                        srcv.at[nb], esem).wait()
                    pltpu.make_async_copy(
                        edges_hbm.at[1, pl.ds(row + SG, SG)],
                        dstv.at[nb], esem).wait()

                scps = []
                for j in range(SG):
                    pltpu.make_async_copy(table.at[srcv.at[b, j]],
                                          msg.at[j], gsem).wait()
                    scps.append(pltpu.async_copy(
                        msg.at[j], acc.at[dstv.at[b, j]], ssem, add=True))

                @pl.when(g + SG < MAIN_ROWS)
                def _():
                    for j in range(SG):
                        scps[j].wait()
                        pltpu.async_copy(table.at[srcv.at[nb, j]],
                                         msg.at[j], gsem)

                @pl.when(g + SG >= MAIN_ROWS)
                def _():
                    for j in range(SG):
                        scps[j].wait()

                @pl.when(g + 2 * SG < MAIN_ROWS)
                def _():
                    pltpu.async_copy(edges_hbm.at[0, pl.ds(row + 2 * SG, SG)],
                                     srcv.at[b], esem)
                    pltpu.async_copy(edges_hbm.at[1, pl.ds(row + 2 * SG, SG)],
                                     dstv.at[b], esem)

            @pl.loop(start + MAIN_ROWS, end)
            def _(row):
                pltpu.sync_copy(edges_hbm.at[0, pl.ds(row, 1)],
                                srcv.at[0, pl.ds(0, 1)])
                pltpu.sync_copy(edges_hbm.at[1, pl.ds(row, 1)],
                                dstv.at[0, pl.ds(0, 1)])
                pltpu.async_copy(table.at[srcv.at[0, 0]], msg.at[0],
                                 gsem).wait()
                pltpu.sync_copy(msg.at[0], acc.at[dstv.at[0, 0]], add=True)

            plsc.subcore_barrier()

            pltpu.sync_copy(acc.at[pl.ds(r0, rpt)],
                            dump_plane.at[pl.ds(r0, rpt)])

        @pl.when(cid == 0)
        def _():
            run(wm_rel_hbm, edges_rel_hbm, dump_hbm.at[0])

        @pl.when(cid == 1)
        def _():
            run(wm_rev_hbm, edges_rev_hbm, dump_hbm.at[1])

    return sc_kernel(wm_rel, wm_rev, edges_rel, edges_rev, zeros)


def _sc_mean(acc_flat):
    mesh = plsc.VectorSubcoreMesh(core_axis_name="c", subcore_axis_name="s")
    rpt = N_PAD // NS
    ovr = rpt * C // 16

    @functools.partial(
        pl.kernel,
        out_type=jax.ShapeDtypeStruct((2, N * C // 16, 16), jnp.float32),
        mesh=mesh,
        compiler_params=pltpu.CompilerParams(use_tc_tiling_on_sc=False),
        scratch_types=[
            pltpu.VMEM((rpt * W,), jnp.float32),
            pltpu.VMEM((ovr, 16), jnp.float32),
        ])
    def mean_kernel(acc_hbm, out_hbm, av, ov):
        cid = lax.axis_index("c")
        sid = lax.axis_index("s")
        i16 = lax.iota(jnp.int32, 16)
        dpat = (i16 >> 3) * 8 + 2
        cpat = ((i16 >> 1) & 1) * 8 + (i16 & 1)

        def run(plane, out_plane):
            r0 = sid * rpt
            pltpu.sync_copy(plane.at[pl.ds(r0 * W, rpt * W)], av)

            @pl.loop(0, rpt * W, step=4 * 16)
            def _(i):
                q = []
                for k in range(4):
                    v = av[pl.ds(i + 16 * k, 16)]
                    dvec = _vtake(v, dpat)
                    qk = jnp.where(dvec > 0.0,
                                   v / jnp.maximum(dvec, 1.0), 0.0)
                    q.append(_vtake(qk, cpat))
                out = jnp.where(i16 < 4, q[0],
                                jnp.where(i16 < 8, q[1],
                                          jnp.where(i16 < 12, q[2], q[3])))
                ov[i >> 6] = out

            last = (N - (NS - 1) * rpt) * C // 16
            o0 = r0 * C // 16

            @pl.when(sid < NS - 1)
            def _():
                pltpu.sync_copy(ov, out_plane.at[pl.ds(o0, ovr)])

            @pl.when(sid == NS - 1)
            def _():
                pltpu.sync_copy(ov.at[pl.ds(0, last)],
                                out_plane.at[pl.ds(o0, last)])

        @pl.when(cid == 0)
        def _():
            run(acc_hbm.at[0], out_hbm.at[1])

        @pl.when(cid == 1)
        def _():
            run(acc_hbm.at[1], out_hbm.at[0])

    return mean_kernel(acc_flat)


def kernel(feat_user, feat_item, edge_index_rel, edge_index_rev,
           W_rel, b_rel, W_rev, b_rev):
    wz = jnp.zeros((D, W - C), jnp.float32)
    tail = jnp.concatenate([jnp.ones((1,), jnp.float32),
                            jnp.zeros((W - C - 1,), jnp.float32)])
    w8_rel = jnp.concatenate([W_rel, wz], axis=1)
    w8_rev = jnp.concatenate([W_rev, wz], axis=1)
    b8_rel = jnp.concatenate([b_rel, tail]).reshape(1, W)
    b8_rev = jnp.concatenate([b_rev, tail]).reshape(1, W)

    wm_rel = _project(feat_user, w8_rel, b8_rel)
    wm_rev = _project(feat_item, w8_rev, b8_rev)

    edges_rel = edge_index_rel.reshape(2, R_TOT, LANE)
    edges_rev = edge_index_rev.reshape(2, R_TOT, LANE)

    zeros = jnp.zeros((N_PAD, W), jnp.float32)
    dump = _sc_aggregate(wm_rel, wm_rev, edges_rel, edges_rev, zeros)
    out = _sc_mean(dump.reshape(2, N_PAD * W))
    return out.reshape(2, N, C)
